# baseline (speedup 1.0000x reference)
import functools

import jax
import jax.numpy as jnp
from jax import lax
from jax.experimental import pallas as pl
from jax.experimental.pallas import tpu as pltpu
from jax.experimental.pallas import tpu_sc as plsc

_ROWS = 128
_N = 32768
_L = 16
_NC = 2
_NS = 16
_NW = _NC * _NS
_RPW = _ROWS // _NW
_UNROLL = 8
_NEG = -3.0e38
_BISECT_ITERS = 40

_mesh = plsc.VectorSubcoreMesh(core_axis_name="c", subcore_axis_name="s")


@functools.partial(
    pl.kernel,
    out_type=jax.ShapeDtypeStruct((_ROWS, _N), jnp.float32),
    mesh=_mesh,
    scratch_types=[
        pltpu.VMEM((_N,), jnp.float32),
        pltpu.VMEM((_N + _L,), jnp.float32),
    ],
    compiler_params=pltpu.CompilerParams(needs_layout_passes=False),
)
def _sc_sparsemax(x_hbm, o_hbm, row_v, cand_v):
    wid = lax.axis_index("s") * _NC + lax.axis_index("c")

    for r in range(_RPW):
        row = wid * _RPW + r
        pltpu.sync_copy(x_hbm.at[row], row_v)

        def mc_body(i, carry):
            mrun, off = carry
            base = i * (_L * _UNROLL)
            for u in range(_UNROLL):
                v = row_v[pl.ds(base + u * _L, _L)]
                mask = v > mrun - 1.0
                mi = mask.astype(jnp.int32)
                c = jnp.cumsum(mi)
                idx = jnp.where(mask, off + c - 1, 0)
                plsc.store_scatter(cand_v, [idx], v, mask=mask)
                off = off + jnp.sum(mi)
                mrun = jnp.maximum(mrun, v)
            return mrun, off

        mrun, off = lax.fori_loop(
            0, _N // (_L * _UNROLL), mc_body,
            (jnp.full((_L,), _NEG, jnp.float32), jnp.int32(0)),
        )
        m = jnp.max(mrun)
        pad_idx = off + lax.iota(jnp.int32, _L)
        plsc.store_scatter(cand_v, [pad_idx], jnp.full((_L,), _NEG, jnp.float32))

        nv = (off + _L - 1) // _L
        lo = m - 1.0
        hi = m

        def bis(_, carry):
            lo, hi = carry
            mid = 0.5 * (lo + hi)

            def fb(i, acc):
                v = cand_v[pl.ds(i * _L, _L)]
                return acc + jnp.maximum(v - mid, 0.0)

            acc = lax.fori_loop(0, nv, fb, jnp.zeros((_L,), jnp.float32))
            f = jnp.sum(acc)
            ge = f >= 1.0
            return jnp.where(ge, mid, lo), jnp.where(ge, hi, mid)

        lo, hi = lax.fori_loop(0, _BISECT_ITERS, bis, (lo, hi))
        tau = lo

        def ob(i, carry):
            base = i * (_L * _UNROLL)
            for u in range(_UNROLL):
                sl = pl.ds(base + u * _L, _L)
                row_v[sl] = jnp.maximum(row_v[sl] - tau, 0.0)
            return carry

        lax.fori_loop(0, _N // (_L * _UNROLL), ob, jnp.int32(0))
        pltpu.sync_copy(row_v, o_hbm.at[row])


@jax.jit
def kernel(logits):
    return _sc_sparsemax(logits.astype(jnp.float32))


# speedup vs baseline: 1.6828x; 1.6828x over previous
import functools

import jax
import jax.numpy as jnp
from jax import lax
from jax.experimental import pallas as pl
from jax.experimental.pallas import tpu as pltpu
from jax.experimental.pallas import tpu_sc as plsc

_ROWS = 128
_N = 32768
_L = 16
_NC = 2
_NS = 16
_NW = _NC * _NS
_RPW = _ROWS // _NW
_UNROLL = 8
_NEG = -3.0e38
_BISECT_ITERS = 32

_mesh = plsc.VectorSubcoreMesh(core_axis_name="c", subcore_axis_name="s")


@functools.partial(
    pl.kernel,
    out_type=jax.ShapeDtypeStruct((_ROWS, _N), jnp.float32),
    mesh=_mesh,
    scratch_types=[
        pltpu.VMEM((_N,), jnp.float32),
        pltpu.VMEM((_N,), jnp.float32),
        pltpu.VMEM((_N + _L,), jnp.float32),
        pltpu.SemaphoreType.DMA,
        pltpu.SemaphoreType.DMA,
        pltpu.SemaphoreType.DMA,
        pltpu.SemaphoreType.DMA,
    ],
    compiler_params=pltpu.CompilerParams(needs_layout_passes=False),
)
def _sc_sparsemax(x_hbm, o_hbm, row_a, row_b, cand_v,
                  sin_a, sin_b, sout_a, sout_b):
    wid = lax.axis_index("s") * _NC + lax.axis_index("c")
    base_row = wid * _RPW
    bufs = (row_a, row_b)
    sins = (sin_a, sin_b)
    souts = (sout_a, sout_b)

    h_in = [None] * _RPW
    h_out = [None] * _RPW
    h_in[0] = pltpu.async_copy(x_hbm.at[base_row], row_a, sin_a)

    for r in range(_RPW):
        p = r % 2
        row_v = bufs[p]
        h_in[r].wait()

        def max_body(i, mrun):
            base = i * (_L * _UNROLL)
            for u in range(_UNROLL):
                mrun = jnp.maximum(mrun, row_v[pl.ds(base + u * _L, _L)])
            return mrun

        mrun = lax.fori_loop(0, _N // (_L * _UNROLL), max_body,
                             jnp.full((_L,), _NEG, jnp.float32))
        m = jnp.max(mrun)
        lo = m - 1.0
        hi = m

        if r >= 1:
            h_out[r - 1].wait()
        if r + 1 < _RPW:
            q = (r + 1) % 2
            h_in[r + 1] = pltpu.async_copy(
                x_hbm.at[base_row + r + 1], bufs[q], sins[q])

        def cp_body(i, off):
            base = i * (_L * _UNROLL)
            for u in range(_UNROLL):
                v = row_v[pl.ds(base + u * _L, _L)]
                mask = v > lo
                pc = plsc.all_reduce_population_count(mask)
                plsc.store_compressed(cand_v.at[pl.ds(off, _L)], v, mask=mask)
                off = off + pc[0]
            return off

        off = lax.fori_loop(0, _N // (_L * _UNROLL), cp_body, jnp.int32(0))
        cand_v[pl.ds(off, _L)] = jnp.full((_L,), _NEG, jnp.float32)
        nv = (off + _L - 1) // _L

        def bis(_, carry):
            lo, hi = carry
            mid = 0.5 * (lo + hi)

            def fb(i, acc):
                v = cand_v[pl.ds(i * _L, _L)]
                return acc + jnp.maximum(v - mid, 0.0)

            acc = lax.fori_loop(0, nv, fb, jnp.zeros((_L,), jnp.float32))
            f = jnp.sum(acc)
            ge = f >= 1.0
            return jnp.where(ge, mid, lo), jnp.where(ge, hi, mid)

        lo, hi = lax.fori_loop(0, _BISECT_ITERS, bis, (lo, hi))
        tau = lo

        def ob(i, carry):
            base = i * (_L * _UNROLL)
            for u in range(_UNROLL):
                sl = pl.ds(base + u * _L, _L)
                row_v[sl] = jnp.maximum(row_v[sl] - tau, 0.0)
            return carry

        lax.fori_loop(0, _N // (_L * _UNROLL), ob, jnp.int32(0))
        h_out[r] = pltpu.async_copy(row_v, o_hbm.at[base_row + r], souts[p])

    h_out[_RPW - 1].wait()


@jax.jit
def kernel(logits):
    return _sc_sparsemax(logits.astype(jnp.float32))
